# Initial kernel scaffold; baseline (speedup 1.0000x reference)
import jax
import jax.numpy as jnp
from jax.experimental import pallas as pl


def kernel(x, emb0, emb1, inter_initial, ae0_w0, ae0_w1, ae0_rb0, ae0_rb1, ae1_w0, ae1_w1, ae1_rb0, ae1_rb1, rec0_w, rec0_b, rec1_w, rec1_b):
    raise NotImplementedError("write your pallas kernel here")


# speedup vs baseline: 4.8802x; 4.8802x over previous
import functools

import jax
import jax.numpy as jnp
from jax import lax
from jax.experimental import pallas as pl
from jax.experimental.pallas import tpu as pltpu


def _tables_kernel(emb0_ref, emb1_ref, inter_ref, w00_ref, w01_ref,
                   w10_ref, w11_ref, rw_ref, rb_ref, a_ref, b_ref):
    h0 = jnp.tanh(lax.dot_general(emb0_ref[...], w00_ref[...],
                                  (((1,), (1,)), ((), ())),
                                  preferred_element_type=jnp.float32))
    t0 = lax.dot_general(h0, w01_ref[...], (((1,), (1,)), ((), ())),
                         preferred_element_type=jnp.float32)
    a_ref[...] = jnp.concatenate([t0, jnp.zeros_like(t0)], axis=1)

    h1 = jnp.tanh(lax.dot_general(emb1_ref[...], w10_ref[...],
                                  (((1,), (1,)), ((), ())),
                                  preferred_element_type=jnp.float32))
    t1 = lax.dot_general(h1, w11_ref[...], (((1,), (1,)), ((), ())),
                         preferred_element_type=jnp.float32)

    f = jnp.tanh(t1)
    recon = lax.dot_general(f, rw_ref[...], (((1,), (1,)), ((), ())),
                            preferred_element_type=jnp.float32) + rb_ref[...]
    d = inter_ref[...].astype(jnp.float32) - recon
    lrow = jnp.mean(d * d, axis=-1, keepdims=True)

    lane = lax.broadcasted_iota(jnp.int32, t1.shape, 1)
    chunk2 = jnp.where(lane == 0, lrow,
                       jnp.where(lane == 1, jnp.float32(1.0), jnp.float32(0.0)))
    b_ref[...] = jnp.concatenate([t1, chunk2], axis=1)


def _gather_kernel(x_sref, tl_ref, out_ref, acc_ref, *, blk, d):
    base = pl.program_id(0) * blk
    acc = jnp.zeros((1, d), jnp.float32)
    for j in range(blk):
        v = x_sref[base + j]
        row = tl_ref[v]
        out_ref[j] = row[:, :d]
        acc = acc + row[:, d:]
    acc_ref[...] = acc.reshape(1, 1, d)


def kernel(x, emb0, emb1, inter_initial,
           ae0_w0, ae0_w1, ae0_rb0, ae0_rb1,
           ae1_w0, ae1_w1, ae1_rb0, ae1_rb1,
           rec0_w, rec0_b, rec1_w, rec1_b):
    B = x.shape[0]
    C0, K = emb0.shape
    C1 = emb1.shape[0]
    D = ae0_w1.shape[0]
    span = rec0_w.shape[0]

    TBLK = min(256, C1)
    grid1 = C1 // TBLK

    tbl_a, tbl_b = pl.pallas_call(
        _tables_kernel,
        grid=(grid1,),
        in_specs=[
            pl.BlockSpec((TBLK, K), lambda i: (i, 0)),
            pl.BlockSpec((TBLK, K), lambda i: (i, 0)),
            pl.BlockSpec((TBLK, span), lambda i: (C0 // TBLK + i, 0)),
            pl.BlockSpec((D, K), lambda i: (0, 0)),
            pl.BlockSpec((D, D), lambda i: (0, 0)),
            pl.BlockSpec((D, K), lambda i: (0, 0)),
            pl.BlockSpec((D, D), lambda i: (0, 0)),
            pl.BlockSpec((span, D), lambda i: (0, 0)),
            pl.BlockSpec((1, span), lambda i: (0, 0)),
        ],
        out_shape=(jax.ShapeDtypeStruct((C0, 2 * D), jnp.float32),
                   jax.ShapeDtypeStruct((C1, 2 * D), jnp.float32)),
        out_specs=(pl.BlockSpec((TBLK, 2 * D), lambda i: (i, 0)),
                   pl.BlockSpec((TBLK, 2 * D), lambda i: (i, 0))),
        compiler_params=pltpu.CompilerParams(
            dimension_semantics=("parallel",)),
    )(emb0, emb1, inter_initial, ae0_w0, ae0_w1, ae1_w0, ae1_w1,
      rec0_w, rec0_b.reshape(1, span))

    n_tab = 1 + C0 + C1
    tl = jnp.concatenate(
        [jnp.zeros((1, 2 * D), jnp.float32), tbl_a, tbl_b], axis=0)
    tl3 = tl.reshape(n_tab, 1, 2 * D)

    BLK = min(256, B)
    grid2 = B // BLK
    grid_spec = pltpu.PrefetchScalarGridSpec(
        num_scalar_prefetch=1,
        grid=(grid2,),
        in_specs=[pl.BlockSpec((n_tab, 1, 2 * D), lambda g, xs: (0, 0, 0))],
        out_specs=[pl.BlockSpec((BLK, 1, D), lambda g, xs: (g, 0, 0)),
                   pl.BlockSpec((1, 1, D), lambda g, xs: (g, 0, 0))],
    )
    out, accs = pl.pallas_call(
        functools.partial(_gather_kernel, blk=BLK, d=D),
        grid_spec=grid_spec,
        out_shape=(jax.ShapeDtypeStruct((B, 1, D), jnp.float32),
                   jax.ShapeDtypeStruct((grid2, 1, D), jnp.float32)),
        compiler_params=pltpu.CompilerParams(
            dimension_semantics=("parallel",)),
    )(x, tl3)

    final = out.reshape(B, D)
    lsum = jnp.sum(accs[:, 0, 0])
    cnt = jnp.sum(accs[:, 0, 1])
    loss = jnp.where(cnt > 0, lsum / jnp.maximum(cnt, 1.0), 0.0) * 100.0
    return final, jnp.reshape(loss, (1,))
